# baseline (speedup 1.0000x reference)
import jax
import jax.numpy as jnp
from jax import lax
from jax.experimental import pallas as pl
from jax.experimental.pallas import tpu as pltpu
from jax.experimental.pallas import tpu_sc as plsc

N_STIMULI = 21
N_DIM = 3
N_REF = 4
BETA = 10.0
GAMMA = 0.001

NC, NS, L = 2, 16, 16
NW = NC * NS
B = 16384
BPW = B // NW
ROW = N_REF + 1
N_PAIR = N_STIMULI * N_STIMULI
PAIR_STEPS = (N_PAIR + L - 1) // L
ROW_STEPS = BPW // L


def _rsqrt(x):
    i = lax.bitcast_convert_type(x, jnp.int32)
    i = 0x5F3759DF - lax.shift_right_logical(i, 1)
    y = lax.bitcast_convert_type(i, jnp.float32)
    for _ in range(3):
        y = y * (1.5 - 0.5 * x * y * y)
    return y


def _body(stimt_hbm, aux_hbm, outpt_hbm, outrt_hbm,
          stim_v, aux_v, tbl_v, prob_v, rt_v, sem):
    wid = lax.axis_index("s") * NC + lax.axis_index("c")
    base = wid * BPW

    stim_dma = pltpu.async_copy(
        stimt_hbm.at[:, pl.ds(base, BPW)], stim_v, sem)
    pltpu.sync_copy(aux_hbm, aux_v)

    lanes = lax.iota(jnp.int32, L)
    zero = lanes - lanes

    def tbl_step(i, _):
        p = jnp.minimum(lanes + i * L, N_PAIR - 1)
        a = p // N_STIMULI
        b = p - a * N_STIMULI
        d2 = jnp.zeros((L,), jnp.float32)
        for k in range(N_DIM):
            ea = plsc.load_gather(aux_v, [zero + k, a])
            eb = plsc.load_gather(aux_v, [zero + k, b])
            df = ea - eb
            d2 = d2 + df * df
        d2 = jnp.maximum(d2, 1e-12)
        d = d2 * _rsqrt(d2)
        tbl_v[pl.ds(i * L, L)] = jnp.exp(-BETA * d) + GAMMA
        return 0

    lax.fori_loop(0, PAIR_STEPS, tbl_step, 0)
    stim_dma.wait()

    def row_step(i, _):
        q = stim_v[0, pl.ds(i * L, L)]
        sims = []
        total = jnp.zeros((L,), jnp.float32)
        for j in range(1, ROW):
            r = stim_v[j, pl.ds(i * L, L)]
            s = plsc.load_gather(tbl_v, [q * N_STIMULI + r])
            s = jnp.where(r != 0, s, 0.0)
            sims.append(s)
            total = total + s
        inv = 1.0 / jnp.maximum(total, 1e-16)
        rt = jnp.zeros((L,), jnp.float32)
        for j, s in enumerate(sims):
            pj = s * inv
            prob_v[j, pl.ds(i * L, L)] = pj
            rt = rt + pj
        rt_v[pl.ds(i * L, L)] = rt
        return 0

    lax.fori_loop(0, ROW_STEPS, row_step, 0)

    pltpu.sync_copy(prob_v, outpt_hbm.at[:, pl.ds(base, BPW)])
    pltpu.sync_copy(rt_v, outrt_hbm.at[pl.ds(base, BPW)])


@jax.jit
def _run(stimt, aux):
    k = pl.kernel(
        _body,
        out_type=(
            jax.ShapeDtypeStruct((N_REF, B), jnp.float32),
            jax.ShapeDtypeStruct((B,), jnp.float32),
        ),
        mesh=plsc.VectorSubcoreMesh(core_axis_name="c", subcore_axis_name="s"),
        compiler_params=pltpu.CompilerParams(needs_layout_passes=False),
        scratch_types=[
            pltpu.VMEM((ROW, BPW), jnp.int32),
            pltpu.VMEM((N_DIM, N_STIMULI), jnp.float32),
            pltpu.VMEM((PAIR_STEPS * L,), jnp.float32),
            pltpu.VMEM((N_REF, BPW), jnp.float32),
            pltpu.VMEM((BPW,), jnp.float32),
            pltpu.SemaphoreType.DMA,
        ],
    )
    return k(stimt, aux)


def kernel(rank_similarity_stimulus_set, embedding, w):
    stimt = rank_similarity_stimulus_set.astype(jnp.int32).T
    aux = (embedding.astype(jnp.float32)
           * jnp.sqrt(w.astype(jnp.float32))[None, :]).T
    probt, rt = _run(stimt, aux)
    return probt.T, rt.reshape(B, 1)


# speedup vs baseline: 1.0535x; 1.0535x over previous
import jax
import jax.numpy as jnp
from jax import lax
from jax.experimental import pallas as pl
from jax.experimental.pallas import tpu as pltpu
from jax.experimental.pallas import tpu_sc as plsc

N_STIMULI = 21
N_DIM = 3
N_REF = 4
BETA = 10.0
GAMMA = 0.001

NC, NS, L = 2, 16, 16
NW = NC * NS
B = 16384
BPW = B // NW
ROW = N_REF + 1
N_PAIR = N_STIMULI * N_STIMULI
PAIR_STEPS = (N_PAIR + L - 1) // L
ROW_STEPS = BPW // L


def _rsqrt(x):
    i = lax.bitcast_convert_type(x, jnp.int32)
    i = 0x5F3759DF - lax.shift_right_logical(i, 1)
    y = lax.bitcast_convert_type(i, jnp.float32)
    for _ in range(3):
        y = y * (1.5 - 0.5 * x * y * y)
    return y


def _body(stimt_hbm, aux_hbm, outpt_hbm, outrt_hbm,
          stim_v, aux_v, tbl_v, prob_v, rt_v, sem):
    wid = lax.axis_index("s") * NC + lax.axis_index("c")
    base = wid * BPW

    stim_dma = pltpu.async_copy(
        stimt_hbm.at[:, pl.ds(base, BPW)], stim_v, sem)
    pltpu.sync_copy(aux_hbm, aux_v)

    lanes = lax.iota(jnp.int32, L)
    zero = lanes - lanes

    @plsc.parallel_loop(0, PAIR_STEPS * L, step=L, unroll=2)
    def tbl_step(i):
        p = jnp.minimum(lanes + i, N_PAIR - 1)
        a = p // N_STIMULI
        b = p - a * N_STIMULI
        d2 = jnp.zeros((L,), jnp.float32)
        for k in range(N_DIM):
            ea = plsc.load_gather(aux_v, [zero + k, a])
            eb = plsc.load_gather(aux_v, [zero + k, b])
            df = ea - eb
            d2 = d2 + df * df
        d2 = jnp.maximum(d2, 1e-12)
        d = d2 * _rsqrt(d2)
        tbl_v[pl.ds(i, L)] = jnp.exp(-BETA * d) + GAMMA

    stim_dma.wait()

    @plsc.parallel_loop(0, BPW, step=L, unroll=4)
    def row_step(i):
        q = stim_v[0, pl.ds(i, L)]
        sims = []
        total = jnp.zeros((L,), jnp.float32)
        for j in range(1, ROW):
            r = stim_v[j, pl.ds(i, L)]
            s = plsc.load_gather(tbl_v, [q * N_STIMULI + r])
            s = jnp.where(r != 0, s, 0.0)
            sims.append(s)
            total = total + s
        inv = 1.0 / jnp.maximum(total, 1e-16)
        rt = jnp.zeros((L,), jnp.float32)
        for j, s in enumerate(sims):
            pj = s * inv
            prob_v[j, pl.ds(i, L)] = pj
            rt = rt + pj
        rt_v[pl.ds(i, L)] = rt

    pltpu.sync_copy(prob_v, outpt_hbm.at[:, pl.ds(base, BPW)])
    pltpu.sync_copy(rt_v, outrt_hbm.at[pl.ds(base, BPW)])


@jax.jit
def _run(stimt, aux):
    k = pl.kernel(
        _body,
        out_type=(
            jax.ShapeDtypeStruct((N_REF, B), jnp.float32),
            jax.ShapeDtypeStruct((B,), jnp.float32),
        ),
        mesh=plsc.VectorSubcoreMesh(core_axis_name="c", subcore_axis_name="s"),
        compiler_params=pltpu.CompilerParams(needs_layout_passes=False),
        scratch_types=[
            pltpu.VMEM((ROW, BPW), jnp.int32),
            pltpu.VMEM((N_DIM, N_STIMULI), jnp.float32),
            pltpu.VMEM((PAIR_STEPS * L,), jnp.float32),
            pltpu.VMEM((N_REF, BPW), jnp.float32),
            pltpu.VMEM((BPW,), jnp.float32),
            pltpu.SemaphoreType.DMA,
        ],
    )
    return k(stimt, aux)


def kernel(rank_similarity_stimulus_set, embedding, w):
    stimt = rank_similarity_stimulus_set.astype(jnp.int32).T
    aux = (embedding.astype(jnp.float32)
           * jnp.sqrt(w.astype(jnp.float32))[None, :]).T
    probt, rt = _run(stimt, aux)
    return probt.T, rt.reshape(B, 1)


# speedup vs baseline: 1.0548x; 1.0012x over previous
import jax
import jax.numpy as jnp
from jax import lax
from jax.experimental import pallas as pl
from jax.experimental.pallas import tpu as pltpu
from jax.experimental.pallas import tpu_sc as plsc

N_STIMULI = 21
N_DIM = 3
N_REF = 4
BETA = 10.0
GAMMA = 0.001

NC, NS, L = 2, 16, 16
NW = NC * NS
B = 16384
BPW = B // NW
ROW = N_REF + 1
N_PAIR = N_STIMULI * N_STIMULI
PAIR_STEPS = (N_PAIR + L - 1) // L
ROW_STEPS = BPW // L


def _rsqrt(x):
    i = lax.bitcast_convert_type(x, jnp.int32)
    i = 0x5F3759DF - lax.shift_right_logical(i, 1)
    y = lax.bitcast_convert_type(i, jnp.float32)
    for _ in range(3):
        y = y * (1.5 - 0.5 * x * y * y)
    return y


def _body(stimt_hbm, aux_hbm, outpt_hbm, outrt_hbm,
          stim_v, aux_v, tbl_v, prob_v, rt_v, sem):
    wid = lax.axis_index("s") * NC + lax.axis_index("c")
    base = wid * BPW

    stim_dma = pltpu.async_copy(
        stimt_hbm.at[:, pl.ds(base, BPW)], stim_v, sem)
    pltpu.sync_copy(aux_hbm, aux_v)

    lanes = lax.iota(jnp.int32, L)
    zero = lanes - lanes

    @plsc.parallel_loop(0, PAIR_STEPS * L, step=L, unroll=2)
    def tbl_step(i):
        p = jnp.minimum(lanes + i, N_PAIR - 1)
        a = p // N_STIMULI
        b = p - a * N_STIMULI
        d2 = jnp.zeros((L,), jnp.float32)
        for k in range(N_DIM):
            ea = plsc.load_gather(aux_v, [zero + k, a])
            eb = plsc.load_gather(aux_v, [zero + k, b])
            df = ea - eb
            d2 = d2 + df * df
        d2 = jnp.maximum(d2, 1e-12)
        d = d2 * _rsqrt(d2)
        tbl_v[pl.ds(i, L)] = jnp.exp(-BETA * d) + GAMMA

    stim_dma.wait()

    @plsc.parallel_loop(0, BPW, step=L, unroll=4)
    def row_step(i):
        q = stim_v[0, pl.ds(i, L)]
        sims = []
        total = jnp.zeros((L,), jnp.float32)
        for j in range(1, ROW):
            r = stim_v[j, pl.ds(i, L)]
            s = plsc.load_gather(tbl_v, [q * N_STIMULI + r])
            s = jnp.where(r != 0, s, 0.0)
            sims.append(s)
            total = total + s
        inv = 1.0 / jnp.maximum(total, 1e-16)
        rt = jnp.zeros((L,), jnp.float32)
        for j, s in enumerate(sims):
            pj = s * inv
            prob_v[j, pl.ds(i, L)] = pj
            rt = rt + pj
        rt_v[pl.ds(i, L)] = rt

    pltpu.sync_copy(prob_v, outpt_hbm.at[:, pl.ds(base, BPW)])
    pltpu.sync_copy(rt_v, outrt_hbm.at[pl.ds(base, BPW)])


@jax.jit
def _run(stimt, aux):
    k = pl.kernel(
        _body,
        out_type=(
            jax.ShapeDtypeStruct((N_REF, B), jnp.float32),
            jax.ShapeDtypeStruct((B,), jnp.float32),
        ),
        mesh=plsc.VectorSubcoreMesh(core_axis_name="c", subcore_axis_name="s"),
        compiler_params=pltpu.CompilerParams(
            needs_layout_passes=False, skip_device_barrier=True),
        scratch_types=[
            pltpu.VMEM((ROW, BPW), jnp.int32),
            pltpu.VMEM((N_DIM, N_STIMULI), jnp.float32),
            pltpu.VMEM((PAIR_STEPS * L,), jnp.float32),
            pltpu.VMEM((N_REF, BPW), jnp.float32),
            pltpu.VMEM((BPW,), jnp.float32),
            pltpu.SemaphoreType.DMA,
        ],
    )
    return k(stimt, aux)


def kernel(rank_similarity_stimulus_set, embedding, w):
    stimt = rank_similarity_stimulus_set.astype(jnp.int32).T
    aux = (embedding.astype(jnp.float32)
           * jnp.sqrt(w.astype(jnp.float32))[None, :]).T
    probt, rt = _run(stimt, aux)
    return probt.T, rt.reshape(B, 1)
